# baseline (speedup 1.0000x reference)
import functools

import jax
import jax.numpy as jnp
from jax import lax
from jax.experimental import pallas as pl
from jax.experimental.pallas import tpu as pltpu
from jax.experimental.pallas import tpu_sc as plsc

_VOCAB = 4096
_NW = 32
_ROWS_PER_W = 8192 // _NW
_C = 16
_NCH = _ROWS_PER_W // _C
_L = 16
_VB = _VOCAB // 16

_mesh = plsc.VectorSubcoreMesh(core_axis_name="c", subcore_axis_name="s")
_sc_params = pltpu.CompilerParams(needs_layout_passes=False)


_SIZES = [16, 8] * 10 + [16]
_STARTS = [sum(_SIZES[:k]) for k in range(len(_SIZES))]
_NRAG = len(_SIZES)


@functools.partial(
    pl.kernel,
    mesh=_mesh,
    compiler_params=_sc_params,
    out_type=[
        jax.ShapeDtypeStruct((8192, _VOCAB), jnp.float32),
        jax.ShapeDtypeStruct((_NW, _L), jnp.float32),
    ],
    scratch_types=[
        pltpu.VMEM((_NRAG, _L), jnp.int32),
        pltpu.VMEM((_NRAG, _L), jnp.int32),
        pltpu.VMEM((16, _VOCAB), jnp.float32),
        pltpu.VMEM((8, _VOCAB), jnp.float32),
        pltpu.VMEM((_L,), jnp.float32),
        pltpu.SemaphoreType.DMA,
        pltpu.SemaphoreType.DMA,
        pltpu.SemaphoreType.DMA,
        pltpu.SemaphoreType.DMA,
    ],
)
def _sc_stream(idx_hbm, tgt_hbm, table_hbm, out_hbm, part_hbm,
               idx_v, tgt_v, rows_a, rows_b, acc_v,
               gsem_a, gsem_b, ssem_a, ssem_b):
    wid = lax.axis_index("s") * 2 + lax.axis_index("c")
    base = wid * _ROWS_PER_W
    pltpu.sync_copy(idx_hbm.at[wid], idx_v)
    pltpu.sync_copy(tgt_hbm.at[wid], tgt_v)
    lanes = lax.iota(jnp.int32, _L)

    def bufref(k):
        s = _SIZES[k]
        buf = rows_a if k % 2 == 0 else rows_b
        return buf if s == buf.shape[0] else buf.at[pl.ds(0, s)]

    def gsem(k):
        return gsem_a if k % 2 == 0 else gsem_b

    def ssem(k):
        return ssem_a if k % 2 == 0 else ssem_b

    def idxref(k):
        s = _SIZES[k]
        return idx_v.at[k] if s == _L else idx_v.at[k, pl.ds(0, s)]

    def outref(k):
        return out_hbm.at[pl.ds(base + _STARTS[k], _SIZES[k])]

    def gather(k):
        pltpu.async_copy(table_hbm.at[idxref(k)], bufref(k), gsem(k))

    def gather_wait(k):
        pltpu.make_async_copy(
            table_hbm.at[idxref(k)], bufref(k), gsem(k)).wait()

    def scatter(k):
        pltpu.async_copy(bufref(k), outref(k), ssem(k))

    def scatter_wait(k):
        pltpu.make_async_copy(bufref(k), outref(k), ssem(k)).wait()

    def tval(k):
        s = _SIZES[k]
        buf = rows_a if k % 2 == 0 else rows_b
        t16 = tgt_v[k]
        if s == _L:
            return plsc.load_gather(buf, [lanes, t16])
        valid = lanes < s
        rid = jnp.where(valid, lanes, 0)
        col = jnp.where(valid, t16, 0)
        v = plsc.load_gather(buf, [rid, col])
        return jnp.where(valid, v, 0.0)

    tacc = jnp.zeros((_L,), jnp.float32)
    gather(0)
    for k in range(_NRAG):
        gather_wait(k)
        if k >= 1:
            scatter_wait(k - 1)
        if k + 1 < _NRAG:
            gather(k + 1)
        tacc = tacc + tval(k)
        scatter(k)
    scatter_wait(_NRAG - 1)
    acc_v[...] = tacc
    pltpu.sync_copy(acc_v, part_hbm.at[wid])


def _lse_count_body(idx_ref, tab_ref, s1_ref):
    i = pl.program_id(0)
    x = tab_ref[...]
    m = jnp.max(x, axis=1, keepdims=True)
    lse = jnp.log(jnp.sum(jnp.exp(x - m), axis=1, keepdims=True)) + m
    rows = i * _VB + lax.broadcasted_iota(jnp.int32, (_VB, 1), 0)

    def cbody(j, cnt):
        ids = idx_ref[:, pl.ds(j * 1024, 1024)]
        eq = (ids == rows).astype(jnp.float32)
        return cnt + jnp.sum(eq, axis=1, keepdims=True)

    cnt = lax.fori_loop(0, 8, cbody, jnp.zeros((_VB, 1), jnp.float32))
    contrib = jnp.full((1, _L), jnp.sum(cnt * lse))

    @pl.when(i == 0)
    def _():
        s1_ref[...] = jnp.zeros((1, _L), jnp.float32)

    s1_ref[...] += contrib


@functools.partial(
    pl.kernel,
    mesh=_mesh,
    compiler_params=_sc_params,
    out_type=jax.ShapeDtypeStruct((1, _L), jnp.float32),
    scratch_types=[
        pltpu.VMEM((1, _L), jnp.float32),
        pltpu.VMEM((_NW, _L), jnp.float32),
        pltpu.VMEM((1, _L), jnp.float32),
    ],
)
def _sc_finalize(s1_hbm, tpart_hbm, out_hbm, s1_v, tpart_v, res_v):
    wid = lax.axis_index("s") * 2 + lax.axis_index("c")

    @pl.when(wid == 0)
    def _():
        pltpu.sync_copy(s1_hbm, s1_v)
        pltpu.sync_copy(tpart_hbm, tpart_v)

        def body(i, acc):
            return acc + tpart_v[i]

        tsum = lax.fori_loop(0, _NW, body, jnp.zeros((_L,), jnp.float32))
        loss = (s1_v[0] - jnp.full((_L,), jnp.sum(tsum))) * (1.0 / 8192.0)
        res_v[0] = loss
        pltpu.sync_copy(res_v, out_hbm)


def _ragged_pad(x):
    xw = x.reshape(_NW, _ROWS_PER_W)
    chunks = []
    for k in range(_NRAG):
        c = xw[:, _STARTS[k]:_STARTS[k] + _SIZES[k]]
        if _SIZES[k] < _L:
            c = jnp.pad(c, ((0, 0), (0, _L - _SIZES[k])))
        chunks.append(c)
    return jnp.stack(chunks, axis=1)


def kernel(idx, targets, table):
    idx_c = _ragged_pad(idx)
    tgt_c = _ragged_pad(targets)
    idx_row = idx.reshape(1, 8192)

    s1 = pl.pallas_call(
        _lse_count_body,
        grid=(16,),
        in_specs=[
            pl.BlockSpec((1, 8192), lambda i: (0, 0)),
            pl.BlockSpec((_VB, _VOCAB), lambda i: (i, 0)),
        ],
        out_specs=pl.BlockSpec((1, _L), lambda i: (0, 0)),
        out_shape=jax.ShapeDtypeStruct((1, _L), jnp.float32),
    )(idx_row, table)

    logits_flat, tpart = _sc_stream(idx_c, tgt_c, table)

    loss = _sc_finalize(s1, tpart)[0, 0]

    return (logits_flat.reshape(idx.shape[0], idx.shape[1], _VOCAB), loss)


# speedup vs baseline: 1.0220x; 1.0220x over previous
import functools

import jax
import jax.numpy as jnp
from jax import lax
from jax.experimental import pallas as pl
from jax.experimental.pallas import tpu as pltpu
from jax.experimental.pallas import tpu_sc as plsc

_VOCAB = 4096
_NW = 32
_ROWS_PER_W = 8192 // _NW
_C = 16
_NCH = _ROWS_PER_W // _C
_L = 16
_VB = _VOCAB // 16

_mesh = plsc.VectorSubcoreMesh(core_axis_name="c", subcore_axis_name="s")
_sc_params = pltpu.CompilerParams(needs_layout_passes=False,
                                  has_side_effects=False)


_SIZES = [16, 8] * 10 + [16]
_STARTS = [sum(_SIZES[:k]) for k in range(len(_SIZES))]
_NRAG = len(_SIZES)


@functools.partial(
    pl.kernel,
    mesh=_mesh,
    compiler_params=_sc_params,
    out_type=[
        jax.ShapeDtypeStruct((8192, _VOCAB), jnp.float32),
        jax.ShapeDtypeStruct((_NW, _L), jnp.float32),
    ],
    scratch_types=[
        pltpu.VMEM((_NRAG, _L), jnp.int32),
        pltpu.VMEM((_NRAG, _L), jnp.int32),
        pltpu.VMEM((16, _VOCAB), jnp.float32),
        pltpu.VMEM((8, _VOCAB), jnp.float32),
        pltpu.VMEM((_L,), jnp.float32),
        pltpu.SemaphoreType.DMA,
        pltpu.SemaphoreType.DMA,
        pltpu.SemaphoreType.DMA,
        pltpu.SemaphoreType.DMA,
    ],
)
def _sc_stream(idx_hbm, tgt_hbm, table_hbm, out_hbm, part_hbm,
               idx_v, tgt_v, rows_a, rows_b, acc_v,
               gsem_a, gsem_b, ssem_a, ssem_b):
    wid = lax.axis_index("s") * 2 + lax.axis_index("c")
    base = wid * _ROWS_PER_W
    pltpu.sync_copy(idx_hbm.at[wid], idx_v)
    pltpu.sync_copy(tgt_hbm.at[wid], tgt_v)
    lanes = lax.iota(jnp.int32, _L)

    def bufref(k):
        s = _SIZES[k]
        buf = rows_a if k % 2 == 0 else rows_b
        return buf if s == buf.shape[0] else buf.at[pl.ds(0, s)]

    def gsem(k):
        return gsem_a if k % 2 == 0 else gsem_b

    def ssem(k):
        return ssem_a if k % 2 == 0 else ssem_b

    def idxref(k):
        s = _SIZES[k]
        return idx_v.at[k] if s == _L else idx_v.at[k, pl.ds(0, s)]

    def outref(k):
        return out_hbm.at[pl.ds(base + _STARTS[k], _SIZES[k])]

    def gather(k):
        pltpu.async_copy(table_hbm.at[idxref(k)], bufref(k), gsem(k))

    def gather_wait(k):
        pltpu.make_async_copy(
            table_hbm.at[idxref(k)], bufref(k), gsem(k)).wait()

    def scatter(k):
        pltpu.async_copy(bufref(k), outref(k), ssem(k))

    def scatter_wait(k):
        pltpu.make_async_copy(bufref(k), outref(k), ssem(k)).wait()

    def tval(k):
        s = _SIZES[k]
        buf = rows_a if k % 2 == 0 else rows_b
        t16 = tgt_v[k]
        if s == _L:
            return plsc.load_gather(buf, [lanes, t16])
        valid = lanes < s
        rid = jnp.where(valid, lanes, 0)
        col = jnp.where(valid, t16, 0)
        v = plsc.load_gather(buf, [rid, col])
        return jnp.where(valid, v, 0.0)

    tacc = jnp.zeros((_L,), jnp.float32)
    gather(0)
    for k in range(_NRAG):
        gather_wait(k)
        if k >= 1:
            scatter_wait(k - 1)
        if k + 1 < _NRAG:
            gather(k + 1)
        tacc = tacc + tval(k)
        scatter(k)
    scatter_wait(_NRAG - 1)
    acc_v[...] = tacc
    pltpu.sync_copy(acc_v, part_hbm.at[wid])


def _lse_count_body(idx_ref, tab_ref, s1_ref):
    i = pl.program_id(0)
    x = tab_ref[...]
    m = jnp.max(x, axis=1, keepdims=True)
    lse = jnp.log(jnp.sum(jnp.exp(x - m), axis=1, keepdims=True)) + m
    rows = i * _VB + lax.broadcasted_iota(jnp.int32, (_VB, 1), 0)

    def cbody(j, cnt):
        ids = idx_ref[:, pl.ds(j * 1024, 1024)]
        eq = (ids == rows).astype(jnp.float32)
        return cnt + jnp.sum(eq, axis=1, keepdims=True)

    cnt = lax.fori_loop(0, 8, cbody, jnp.zeros((_VB, 1), jnp.float32))
    contrib = jnp.sum(cnt * lse).reshape(1, 1)

    @pl.when(i == 0)
    def _():
        s1_ref[...] = jnp.zeros((1, 1), jnp.float32)

    s1_ref[...] += contrib


def _finalize_body(s1_ref, tpart_ref, out_ref):
    s = s1_ref[0, 0] - jnp.sum(tpart_ref[...])
    out_ref[...] = jnp.reshape(s * (1.0 / 8192.0), (1, 1))


def _ragged_pad(x):
    xw = x.reshape(_NW, _ROWS_PER_W)
    chunks = []
    for k in range(_NRAG):
        c = xw[:, _STARTS[k]:_STARTS[k] + _SIZES[k]]
        if _SIZES[k] < _L:
            c = jnp.pad(c, ((0, 0), (0, _L - _SIZES[k])))
        chunks.append(c)
    return jnp.stack(chunks, axis=1)


def kernel(idx, targets, table):
    idx_c = _ragged_pad(idx)
    tgt_c = _ragged_pad(targets)
    idx_row = idx.reshape(1, 8192)

    s1 = pl.pallas_call(
        _lse_count_body,
        grid=(16,),
        in_specs=[
            pl.BlockSpec((1, 8192), lambda i: (0, 0)),
            pl.BlockSpec((_VB, _VOCAB), lambda i: (i, 0)),
        ],
        out_specs=pl.BlockSpec((1, 1), lambda i: (0, 0)),
        out_shape=jax.ShapeDtypeStruct((1, 1), jnp.float32),
    )(idx_row, table)

    logits_flat, tpart = _sc_stream(idx_c, tgt_c, table)

    loss = pl.pallas_call(
        _finalize_body,
        out_shape=jax.ShapeDtypeStruct((1, 1), jnp.float32),
    )(s1, tpart)[0, 0]

    return (logits_flat.reshape(idx.shape[0], idx.shape[1], _VOCAB), loss)


# speedup vs baseline: 1.0236x; 1.0015x over previous
import functools

import jax
import jax.numpy as jnp
from jax import lax
from jax.experimental import pallas as pl
from jax.experimental.pallas import tpu as pltpu
from jax.experimental.pallas import tpu_sc as plsc

_VOCAB = 4096
_NW = 32
_ROWS_PER_W = 8192 // _NW
_C = 16
_NCH = _ROWS_PER_W // _C
_L = 16
_VB = _VOCAB // 16

_mesh = plsc.VectorSubcoreMesh(core_axis_name="c", subcore_axis_name="s")
_sc_params = pltpu.CompilerParams(needs_layout_passes=False)


_SIZES = [16, 8] * 10 + [16]
_STARTS = [sum(_SIZES[:k]) for k in range(len(_SIZES))]
_NRAG = len(_SIZES)


@functools.partial(
    pl.kernel,
    mesh=_mesh,
    compiler_params=_sc_params,
    out_type=[
        jax.ShapeDtypeStruct((8192, _VOCAB), jnp.float32),
        jax.ShapeDtypeStruct((_NW, _L), jnp.float32),
    ],
    scratch_types=[
        pltpu.VMEM((_NRAG, _L), jnp.int32),
        pltpu.VMEM((_NRAG, _L), jnp.int32),
        pltpu.VMEM((16, _VOCAB), jnp.float32),
        pltpu.VMEM((8, _VOCAB), jnp.float32),
        pltpu.VMEM((_L,), jnp.float32),
        pltpu.SemaphoreType.DMA,
        pltpu.SemaphoreType.DMA,
        pltpu.SemaphoreType.DMA,
        pltpu.SemaphoreType.DMA,
    ],
)
def _sc_stream(idx_hbm, tgt_hbm, table_hbm, out_hbm, part_hbm,
               idx_v, tgt_v, rows_a, rows_b, acc_v,
               gsem_a, gsem_b, ssem_a, ssem_b):
    wid = lax.axis_index("s") * 2 + lax.axis_index("c")
    base = wid * _ROWS_PER_W
    pltpu.sync_copy(idx_hbm.at[wid], idx_v)
    pltpu.sync_copy(tgt_hbm.at[wid], tgt_v)
    lanes = lax.iota(jnp.int32, _L)

    def bufref(k):
        s = _SIZES[k]
        buf = rows_a if k % 2 == 0 else rows_b
        return buf if s == buf.shape[0] else buf.at[pl.ds(0, s)]

    def gsem(k):
        return gsem_a if k % 2 == 0 else gsem_b

    def ssem(k):
        return ssem_a if k % 2 == 0 else ssem_b

    def idxref(k):
        s = _SIZES[k]
        return idx_v.at[k] if s == _L else idx_v.at[k, pl.ds(0, s)]

    def outref(k):
        return out_hbm.at[pl.ds(base + _STARTS[k], _SIZES[k])]

    def gather(k):
        pltpu.async_copy(table_hbm.at[idxref(k)], bufref(k), gsem(k))

    def gather_wait(k):
        pltpu.make_async_copy(
            table_hbm.at[idxref(k)], bufref(k), gsem(k)).wait()

    def scatter(k):
        pltpu.async_copy(bufref(k), outref(k), ssem(k))

    def scatter_wait(k):
        pltpu.make_async_copy(bufref(k), outref(k), ssem(k)).wait()

    def tval(k):
        s = _SIZES[k]
        buf = rows_a if k % 2 == 0 else rows_b
        t16 = tgt_v[k]
        if s == _L:
            return plsc.load_gather(buf, [lanes, t16])
        valid = lanes < s
        rid = jnp.where(valid, lanes, 0)
        col = jnp.where(valid, t16, 0)
        v = plsc.load_gather(buf, [rid, col])
        return jnp.where(valid, v, 0.0)

    tacc = jnp.zeros((_L,), jnp.float32)
    gather(0)
    for k in range(_NRAG):
        gather_wait(k)
        if k >= 1:
            scatter_wait(k - 1)
        if k + 1 < _NRAG:
            gather(k + 1)
        tacc = tacc + tval(k)
        scatter(k)
    scatter_wait(_NRAG - 1)
    acc_v[...] = tacc
    pltpu.sync_copy(acc_v, part_hbm.at[wid])


def _lse_count_body(idx_ref, tab_ref, s1_ref):
    i = pl.program_id(0)
    x = tab_ref[...]
    m = jnp.max(x, axis=1, keepdims=True)
    lse = jnp.log(jnp.sum(jnp.exp(x - m), axis=1, keepdims=True)) + m
    rows = i * _VB + lax.broadcasted_iota(jnp.int32, (_VB, 1), 0)

    def cbody(j, cnt):
        ids = idx_ref[:, pl.ds(j * 1024, 1024)]
        eq = (ids == rows).astype(jnp.float32)
        return cnt + jnp.sum(eq, axis=1, keepdims=True)

    cnt = lax.fori_loop(0, 8, cbody, jnp.zeros((_VB, 1), jnp.float32))
    contrib = jnp.sum(cnt * lse).reshape(1, 1)

    @pl.when(i == 0)
    def _():
        s1_ref[...] = jnp.zeros((1, 1), jnp.float32)

    s1_ref[...] += contrib


def _finalize_body(s1_ref, tpart_ref, out_ref):
    s = s1_ref[0, 0] - jnp.sum(tpart_ref[...])
    out_ref[...] = jnp.reshape(s * (1.0 / 8192.0), (1, 1))


def _ragged_pad(x):
    xw = x.reshape(_NW, _ROWS_PER_W)
    chunks = []
    for k in range(_NRAG):
        c = xw[:, _STARTS[k]:_STARTS[k] + _SIZES[k]]
        if _SIZES[k] < _L:
            c = jnp.pad(c, ((0, 0), (0, _L - _SIZES[k])))
        chunks.append(c)
    return jnp.stack(chunks, axis=1)


def kernel(idx, targets, table):
    idx_c = _ragged_pad(idx)
    tgt_c = _ragged_pad(targets)
    idx_row = idx.reshape(1, 8192)

    s1 = pl.pallas_call(
        _lse_count_body,
        grid=(16,),
        in_specs=[
            pl.BlockSpec((1, 8192), lambda i: (0, 0)),
            pl.BlockSpec((_VB, _VOCAB), lambda i: (i, 0)),
        ],
        out_specs=pl.BlockSpec((1, 1), lambda i: (0, 0)),
        out_shape=jax.ShapeDtypeStruct((1, 1), jnp.float32),
    )(idx_row, table)

    logits_flat, tpart = _sc_stream(idx_c, tgt_c, table)

    loss = pl.pallas_call(
        _finalize_body,
        out_shape=jax.ShapeDtypeStruct((1, 1), jnp.float32),
    )(s1, tpart)[0, 0]

    return (logits_flat.reshape(idx.shape[0], idx.shape[1], _VOCAB), loss)


# speedup vs baseline: 1.0667x; 1.0422x over previous
import functools

import jax
import jax.numpy as jnp
from jax import lax
from jax.experimental import pallas as pl
from jax.experimental.pallas import tpu as pltpu
from jax.experimental.pallas import tpu_sc as plsc

_VOCAB = 4096
_NW = 32
_ROWS_PER_W = 8192 // _NW
_C = 16
_NCH = _ROWS_PER_W // _C
_L = 16
_VB = 512

_mesh = plsc.VectorSubcoreMesh(core_axis_name="c", subcore_axis_name="s")
_sc_params = pltpu.CompilerParams(needs_layout_passes=False)


_SIZES = [16, 8] * 10 + [16]
_STARTS = [sum(_SIZES[:k]) for k in range(len(_SIZES))]
_NRAG = len(_SIZES)


@functools.partial(
    pl.kernel,
    mesh=_mesh,
    compiler_params=_sc_params,
    out_type=[
        jax.ShapeDtypeStruct((8192, _VOCAB), jnp.float32),
        jax.ShapeDtypeStruct((_NW, _L), jnp.float32),
    ],
    scratch_types=[
        pltpu.VMEM((_ROWS_PER_W,), jnp.int32),
        pltpu.VMEM((_ROWS_PER_W,), jnp.int32),
        pltpu.VMEM((16, _VOCAB), jnp.float32),
        pltpu.VMEM((8, _VOCAB), jnp.float32),
        pltpu.VMEM((_L,), jnp.float32),
        pltpu.SemaphoreType.DMA,
        pltpu.SemaphoreType.DMA,
        pltpu.SemaphoreType.DMA,
        pltpu.SemaphoreType.DMA,
    ],
)
def _sc_stream(idx_hbm, tgt_hbm, table_hbm, out_hbm, part_hbm,
               idx_v, tgt_v, rows_a, rows_b, acc_v,
               gsem_a, gsem_b, ssem_a, ssem_b):
    wid = lax.axis_index("s") * 2 + lax.axis_index("c")
    base = wid * _ROWS_PER_W
    pltpu.sync_copy(idx_hbm.at[wid], idx_v)
    pltpu.sync_copy(tgt_hbm.at[wid], tgt_v)
    lanes = lax.iota(jnp.int32, _L)

    def bufref(k):
        s = _SIZES[k]
        buf = rows_a if k % 2 == 0 else rows_b
        return buf if s == buf.shape[0] else buf.at[pl.ds(0, s)]

    def gsem(k):
        return gsem_a if k % 2 == 0 else gsem_b

    def ssem(k):
        return ssem_a if k % 2 == 0 else ssem_b

    def idxref(k):
        return idx_v.at[pl.ds(_STARTS[k], _SIZES[k])]

    def outref(k):
        return out_hbm.at[pl.ds(base + _STARTS[k], _SIZES[k])]

    def gather(k):
        pltpu.async_copy(table_hbm.at[idxref(k)], bufref(k), gsem(k))

    def gather_wait(k):
        pltpu.make_async_copy(
            table_hbm.at[idxref(k)], bufref(k), gsem(k)).wait()

    def scatter(k):
        pltpu.async_copy(bufref(k), outref(k), ssem(k))

    def scatter_wait(k):
        pltpu.make_async_copy(bufref(k), outref(k), ssem(k)).wait()

    def tval(k):
        s = _SIZES[k]
        buf = rows_a if k % 2 == 0 else rows_b
        t16 = tgt_v[pl.ds(_STARTS[k], _L)]
        if s == _L:
            return plsc.load_gather(buf, [lanes, t16])
        valid = lanes < s
        rid = jnp.where(valid, lanes, 0)
        col = jnp.where(valid, t16, 0)
        v = plsc.load_gather(buf, [rid, col])
        return jnp.where(valid, v, 0.0)

    tacc = jnp.zeros((_L,), jnp.float32)
    gather(0)
    for k in range(_NRAG):
        gather_wait(k)
        if k >= 1:
            scatter_wait(k - 1)
        if k + 1 < _NRAG:
            gather(k + 1)
        tacc = tacc + tval(k)
        scatter(k)
    scatter_wait(_NRAG - 1)
    acc_v[...] = tacc
    pltpu.sync_copy(acc_v, part_hbm.at[wid])


def _lse_count_body(idx_ref, tab_ref, s1_ref):
    i = pl.program_id(0)
    x = tab_ref[...]
    m = jnp.max(x, axis=1, keepdims=True)
    lse = jnp.log(jnp.sum(jnp.exp(x - m), axis=1, keepdims=True)) + m
    rows = i * _VB + lax.broadcasted_iota(jnp.int32, (_VB, 1), 0)

    def cbody(j, cnt):
        ids = idx_ref[:, pl.ds(j * 1024, 1024)]
        eq = (ids == rows).astype(jnp.float32)
        return cnt + jnp.sum(eq, axis=1, keepdims=True)

    cnt = lax.fori_loop(0, 8, cbody, jnp.zeros((_VB, 1), jnp.float32))
    contrib = jnp.sum(cnt * lse).reshape(1, 1)

    @pl.when(i == 0)
    def _():
        s1_ref[...] = jnp.zeros((1, 1), jnp.float32)

    s1_ref[...] += contrib


def _finalize_body(s1_ref, tpart_ref, out_ref):
    s = s1_ref[0, 0] - jnp.sum(tpart_ref[...])
    out_ref[...] = jnp.reshape(s * (1.0 / 8192.0), (1, 1))


def kernel(idx, targets, table):
    idx_c = idx.reshape(_NW, _ROWS_PER_W)
    tgt_c = targets.reshape(_NW, _ROWS_PER_W)
    idx_row = idx.reshape(1, 8192)

    s1 = pl.pallas_call(
        _lse_count_body,
        grid=(_VOCAB // _VB,),
        in_specs=[
            pl.BlockSpec((1, 8192), lambda i: (0, 0)),
            pl.BlockSpec((_VB, _VOCAB), lambda i: (i, 0)),
        ],
        out_specs=pl.BlockSpec((1, 1), lambda i: (0, 0)),
        out_shape=jax.ShapeDtypeStruct((1, 1), jnp.float32),
    )(idx_row, table)

    logits_flat, tpart = _sc_stream(idx_c, tgt_c, table)

    loss = pl.pallas_call(
        _finalize_body,
        out_shape=jax.ShapeDtypeStruct((1, 1), jnp.float32),
    )(s1, tpart)[0, 0]

    return (logits_flat.reshape(idx.shape[0], idx.shape[1], _VOCAB), loss)
